# baseline (speedup 1.0000x reference)
import jax
import jax.numpy as jnp
from jax.experimental import pallas as pl
from jax.experimental.pallas import tpu as pltpu


def _moe_body(ne, bt, bf, x_ref, wg_ref, w1_ref, w2_ref, w3_ref,
              out_ref, logits_ref, wfull_ref):
    e = pl.program_id(0)
    f = pl.program_id(1)
    b = pl.program_id(2)
    rows = pl.ds(b * bt, bt)
    xb = x_ref[rows, :]

    @pl.when((e == 0) & (f == 0))
    def _router():
        logits = jax.lax.dot_general(
            xb, wg_ref[...], (((1,), (1,)), ((), ())),
            preferred_element_type=jnp.float32)
        logits_ref[rows, :] = logits
        m = jnp.max(logits, axis=-1, keepdims=True)
        ex = jnp.exp(logits - m)
        scores = ex / jnp.sum(ex, axis=-1, keepdims=True)
        cols = jax.lax.broadcasted_iota(jnp.int32, scores.shape, 1)
        m1 = jnp.max(scores, axis=-1, keepdims=True)
        idx1 = jnp.min(jnp.where(scores == m1, cols, ne), axis=-1,
                       keepdims=True)
        masked = jnp.where(cols == idx1, -jnp.inf, scores)
        m2 = jnp.max(masked, axis=-1, keepdims=True)
        idx2 = jnp.min(jnp.where(masked == m2, cols, ne), axis=-1,
                       keepdims=True)
        denom = jnp.clip(m1 + m2, 1e-12, None)
        wf = (jnp.where(cols == idx1, m1 / denom, 0.0)
              + jnp.where(cols == idx2, m2 / denom, 0.0))
        wfull_ref[rows, :] = wf.astype(jnp.float32)
        out_ref[rows, :] = jnp.zeros_like(xb)

    xb16 = xb.astype(jnp.bfloat16)
    gate = jax.lax.dot_general(
        xb16, w1_ref[0].astype(jnp.bfloat16), (((1,), (1,)), ((), ())),
        preferred_element_type=jnp.float32)
    up = jax.lax.dot_general(
        xb16, w3_ref[0].astype(jnp.bfloat16), (((1,), (1,)), ((), ())),
        preferred_element_type=jnp.float32)
    act = gate * jax.nn.sigmoid(gate) * up
    yp = jax.lax.dot_general(
        act.astype(jnp.bfloat16), w2_ref[0].astype(jnp.bfloat16),
        (((1,), (1,)), ((), ())),
        preferred_element_type=jnp.float32)
    w8 = wfull_ref[rows, :]
    cols = jax.lax.broadcasted_iota(jnp.int32, w8.shape, 1)
    we = jnp.sum(jnp.where(cols == e, w8, 0.0), axis=-1, keepdims=True)
    out_ref[rows, :] += yp * we


def kernel(hidden_states, W_gate, W1, W2, W3):
    B, S, D = hidden_states.shape
    T = B * S
    E, F = W1.shape[0], W1.shape[1]
    x = hidden_states.reshape(T, D)

    bt = min(256, T)
    bf = min(1024, F)
    nb = T // bt
    nf = F // bf

    body = lambda *refs: _moe_body(E, bt, bf, *refs)
    final, logits = pl.pallas_call(
        body,
        grid=(E, nf, nb),
        in_specs=[
            pl.BlockSpec((T, D), lambda e, f, b: (0, 0)),
            pl.BlockSpec((E, D), lambda e, f, b: (0, 0)),
            pl.BlockSpec((1, bf, D), lambda e, f, b: (e, f, 0)),
            pl.BlockSpec((1, D, bf), lambda e, f, b: (e, 0, f)),
            pl.BlockSpec((1, bf, D), lambda e, f, b: (e, f, 0)),
        ],
        out_specs=[
            pl.BlockSpec((T, D), lambda e, f, b: (0, 0)),
            pl.BlockSpec((T, E), lambda e, f, b: (0, 0)),
        ],
        out_shape=[
            jax.ShapeDtypeStruct((T, D), jnp.float32),
            jax.ShapeDtypeStruct((T, E), jnp.float32),
        ],
        scratch_shapes=[pltpu.VMEM((T, E), jnp.float32)],
        compiler_params=pltpu.CompilerParams(
            dimension_semantics=("arbitrary", "arbitrary", "arbitrary"),
            vmem_limit_bytes=63 * 1024 * 1024,
        ),
    )(x, W_gate, W1, W2, W3)
    return final.reshape(B, S, D), logits


# speedup vs baseline: 1.1894x; 1.1894x over previous
import jax
import jax.numpy as jnp
from jax.experimental import pallas as pl
from jax.experimental.pallas import tpu as pltpu


def _router_body(ne, x_ref, wg_ref, logits_ref, wfull_ref):
    logits = jax.lax.dot_general(
        x_ref[...], wg_ref[...], (((1,), (1,)), ((), ())),
        preferred_element_type=jnp.float32)
    logits_ref[...] = logits
    m = jnp.max(logits, axis=-1, keepdims=True)
    ex = jnp.exp(logits - m)
    scores = ex / jnp.sum(ex, axis=-1, keepdims=True)
    cols = jax.lax.broadcasted_iota(jnp.int32, scores.shape, 1)
    m1 = jnp.max(scores, axis=-1, keepdims=True)
    idx1 = jnp.min(jnp.where(scores == m1, cols, ne), axis=-1, keepdims=True)
    masked = jnp.where(cols == idx1, -jnp.inf, scores)
    m2 = jnp.max(masked, axis=-1, keepdims=True)
    idx2 = jnp.min(jnp.where(masked == m2, cols, ne), axis=-1, keepdims=True)
    denom = jnp.clip(m1 + m2, 1e-12, None)
    wfull_ref[...] = (jnp.where(cols == idx1, m1 / denom, 0.0)
                      + jnp.where(cols == idx2, m2 / denom, 0.0))


def _experts_body(bt, x_ref, w1_ref, w2_ref, w3_ref, wfull_ref, out_ref):
    e = pl.program_id(0)
    f = pl.program_id(1)
    b = pl.program_id(2)
    rows = pl.ds(b * bt, bt)
    xb16 = x_ref[rows, :].astype(jnp.bfloat16)

    @pl.when((e == 0) & (f == 0))
    def _init():
        out_ref[rows, :] = jnp.zeros((bt, out_ref.shape[1]), jnp.float32)

    gate = jax.lax.dot_general(
        xb16, w1_ref[0].astype(jnp.bfloat16), (((1,), (1,)), ((), ())),
        preferred_element_type=jnp.float32)
    up = jax.lax.dot_general(
        xb16, w3_ref[0].astype(jnp.bfloat16), (((1,), (1,)), ((), ())),
        preferred_element_type=jnp.float32)
    act = gate * jax.nn.sigmoid(gate) * up
    yp = jax.lax.dot_general(
        act.astype(jnp.bfloat16), w2_ref[0].astype(jnp.bfloat16),
        (((1,), (1,)), ((), ())),
        preferred_element_type=jnp.float32)
    w8 = wfull_ref[rows, :]
    cols = jax.lax.broadcasted_iota(jnp.int32, w8.shape, 1)
    we = jnp.sum(jnp.where(cols == e, w8, 0.0), axis=-1, keepdims=True)
    out_ref[rows, :] += yp * we


def kernel(hidden_states, W_gate, W1, W2, W3):
    B, S, D = hidden_states.shape
    T = B * S
    E, F = W1.shape[0], W1.shape[1]
    x = hidden_states.reshape(T, D)

    logits, wfull = pl.pallas_call(
        lambda *r: _router_body(E, *r),
        out_shape=[
            jax.ShapeDtypeStruct((T, E), jnp.float32),
            jax.ShapeDtypeStruct((T, E), jnp.float32),
        ],
    )(x, W_gate)

    bt = min(512, T)
    bf = min(1024, F)
    nb = T // bt
    nf = F // bf

    final = pl.pallas_call(
        lambda *r: _experts_body(bt, *r),
        grid=(E, nf, nb),
        in_specs=[
            pl.BlockSpec((T, D), lambda e, f, b: (0, 0)),
            pl.BlockSpec((1, bf, D), lambda e, f, b: (e, f, 0)),
            pl.BlockSpec((1, D, bf), lambda e, f, b: (e, 0, f)),
            pl.BlockSpec((1, bf, D), lambda e, f, b: (e, f, 0)),
            pl.BlockSpec((T, E), lambda e, f, b: (0, 0)),
        ],
        out_specs=pl.BlockSpec((T, D), lambda e, f, b: (0, 0)),
        out_shape=jax.ShapeDtypeStruct((T, D), jnp.float32),
        compiler_params=pltpu.CompilerParams(
            dimension_semantics=("arbitrary", "arbitrary", "arbitrary"),
            vmem_limit_bytes=63 * 1024 * 1024,
        ),
    )(x, W1, W2, W3, wfull)
    return final.reshape(B, S, D), logits


# speedup vs baseline: 1.3637x; 1.1465x over previous
import functools

import jax
import jax.numpy as jnp
from jax import lax
from jax.experimental import pallas as pl
from jax.experimental.pallas import tpu as pltpu
from jax.experimental.pallas import tpu_sc as plsc

E = 8
TOPK = 2
BT = 256
NC = 2


def _router_body(x_ref, wg_ref, logits_ref, sel_ref, wbe_ref, wbo_ref):
    logits = lax.dot_general(
        x_ref[...], wg_ref[...], (((1,), (1,)), ((), ())),
        preferred_element_type=jnp.float32)
    logits_ref[...] = logits
    m = jnp.max(logits, axis=-1, keepdims=True)
    ex = jnp.exp(logits - m)
    scores = ex / jnp.sum(ex, axis=-1, keepdims=True)
    cols = lax.broadcasted_iota(jnp.int32, scores.shape, 1)
    m1 = jnp.max(scores, axis=-1, keepdims=True)
    idx1 = jnp.min(jnp.where(scores == m1, cols, E), axis=-1, keepdims=True)
    masked = jnp.where(cols == idx1, -jnp.inf, scores)
    m2 = jnp.max(masked, axis=-1, keepdims=True)
    idx2 = jnp.min(jnp.where(masked == m2, cols, E), axis=-1, keepdims=True)
    denom = jnp.clip(m1 + m2, 1e-12, None)
    sel_ref[...] = jnp.concatenate([idx1, idx2], axis=1)
    d = wbe_ref.shape[1]
    wbe_ref[...] = lax.broadcast_in_dim(m1 / denom, (m1.shape[0], d), (0, 1))
    wbo_ref[...] = lax.broadcast_in_dim(m2 / denom, (m2.shape[0], d), (0, 1))


def _meta_body(ep_ref, dest_ref, nblk_ref, gsb_ref, carry_s, gs_s):
    ph = pl.program_id(0)
    b = pl.program_id(1)
    epb = ep_ref[...]
    cols = lax.broadcasted_iota(jnp.int32, (BT, E), 1)
    oh = (epb == cols).astype(jnp.float32)

    @pl.when((ph == 0) & (b == 0))
    def _init():
        carry_s[...] = jnp.zeros((1, E), jnp.float32)

    @pl.when(ph == 0)
    def _count():
        carry_s[...] += jnp.sum(oh, axis=0, keepdims=True)

    @pl.when((ph == 1) & (b == 0))
    def _offsets():
        tot = carry_s[...]
        padded = jnp.floor((tot + (BT - 1)) * (1.0 / BT)) * float(BT)
        ei = lax.broadcasted_iota(jnp.int32, (E, E), 0)
        ej = lax.broadcasted_iota(jnp.int32, (E, E), 1)
        excl = (ei < ej).astype(jnp.float32)
        gs_s[...] = jnp.dot(padded, excl,
                            preferred_element_type=jnp.float32)
        nblk_ref[...] = (padded * (1.0 / BT)).astype(jnp.int32)
        gsb_ref[...] = (gs_s[...] * (1.0 / BT)).astype(jnp.int32)
        carry_s[...] = jnp.zeros((1, E), jnp.float32)

    @pl.when(ph == 1)
    def _rank():
        ri = lax.broadcasted_iota(jnp.int32, (BT, BT), 0)
        ci = lax.broadcasted_iota(jnp.int32, (BT, BT), 1)
        tri = (ri > ci).astype(jnp.float32)
        rank = jnp.dot(tri, oh, preferred_element_type=jnp.float32)
        rank = rank + carry_s[...]
        destv = jnp.sum(oh * (gs_s[...] + rank), axis=1, keepdims=True)
        dest_ref[...] = destv.astype(jnp.int32)
        carry_s[...] += jnp.sum(oh, axis=0, keepdims=True)


def _dispatch_body(tok_hbm, dest_hbm, x_hbm, xs_hbm, tok_v, dche_v, rows_v,
                   sem1, sem2):
    wid = lax.axis_index("s") * NC + lax.axis_index("c")
    pbase = wid * 128
    for k in range(4):
        pltpu.sync_copy(tok_hbm.at[pl.ds(pbase + k * 32, 32)], tok_v)
        pltpu.sync_copy(dest_hbm.at[pl.ds(pbase + k * 32, 32)], dche_v)
        pltpu.async_copy(x_hbm.at[tok_v], rows_v, sem1).wait()
        pltpu.async_copy(rows_v, xs_hbm.at[dche_v], sem2).wait()


def _g1_body(nblk_ref, gsb_ref, xs_ref, w1_ref, w3_ref, act_ref):
    e = pl.program_id(0)
    j = pl.program_id(2)

    @pl.when(j < nblk_ref[e])
    def _():
        xb = xs_ref[...]
        gate = lax.dot_general(
            xb, w1_ref[0], (((1,), (1,)), ((), ())),
            preferred_element_type=jnp.float32)
        up = lax.dot_general(
            xb, w3_ref[0], (((1,), (1,)), ((), ())),
            preferred_element_type=jnp.float32)
        act_ref[...] = gate * jax.nn.sigmoid(gate) * up


def _g2_body(nblk_ref, gsb_ref, act_ref, w2_ref, y_ref):
    e = pl.program_id(0)
    j = pl.program_id(1)

    @pl.when(j < nblk_ref[e])
    def _():
        y_ref[...] = lax.dot_general(
            act_ref[...], w2_ref[0], (((1,), (1,)), ((), ())),
            preferred_element_type=jnp.float32)


def _combine_body(y_hbm, de_hbm, do_hbm, wbe_hbm, wbo_hbm, out_hbm,
                  ie_v, io_v, r0_v, r1_v, w0_v, w1_v, acc_v, sem1, sem2):
    wid = lax.axis_index("s") * NC + lax.axis_index("c")
    t0base = wid * 64
    for g in range(4):
        t0 = t0base + g * 16
        pltpu.sync_copy(de_hbm.at[pl.ds(t0, 16)], ie_v)
        pltpu.sync_copy(do_hbm.at[pl.ds(t0, 16)], io_v)
        pltpu.async_copy(y_hbm.at[ie_v], r0_v, sem1).wait()
        pltpu.async_copy(y_hbm.at[io_v], r1_v, sem2).wait()
        pltpu.sync_copy(wbe_hbm.at[pl.ds(t0, 16)], w0_v)
        pltpu.sync_copy(wbo_hbm.at[pl.ds(t0, 16)], w1_v)
        for i in range(16):
            def vbody(vv, carry, i=i):
                s = pl.ds(vv * 16, 16)
                acc_v[i, s] = (w0_v[i, s] * r0_v[i, s]
                               + w1_v[i, s] * r1_v[i, s])
                return carry

            lax.fori_loop(0, 64, vbody, None)
        pltpu.sync_copy(acc_v, out_hbm.at[pl.ds(t0, 16)])


def kernel(hidden_states, W_gate, W1, W2, W3):
    B, S, D = hidden_states.shape
    T = B * S
    F = W1.shape[1]
    x = hidden_states.reshape(T, D)
    npair = T * TOPK
    nbe = T // BT
    nblk_max = npair // BT + E
    trash = nblk_max
    p_alloc = (nblk_max + 1) * BT

    logits, sel, wbe, wbo = pl.pallas_call(
        _router_body,
        out_shape=[
            jax.ShapeDtypeStruct((T, E), jnp.float32),
            jax.ShapeDtypeStruct((T, TOPK), jnp.int32),
            jax.ShapeDtypeStruct((T, D), jnp.float32),
            jax.ShapeDtypeStruct((T, D), jnp.float32),
        ],
    )(x, W_gate)

    ep2d = jnp.concatenate([sel[:, 0:1], sel[:, 1:2]], axis=0)
    dest2d, nblk2d, gsb2d = pl.pallas_call(
        _meta_body,
        grid=(2, npair // BT),
        in_specs=[pl.BlockSpec((BT, 1), lambda ph, b: (b, 0))],
        out_specs=[
            pl.BlockSpec((BT, 1),
                         lambda ph, b: (jnp.where(ph == 1, b, 0), 0)),
            pl.BlockSpec((1, E), lambda ph, b: (0, 0)),
            pl.BlockSpec((1, E), lambda ph, b: (0, 0)),
        ],
        out_shape=[
            jax.ShapeDtypeStruct((npair, 1), jnp.int32),
            jax.ShapeDtypeStruct((1, E), jnp.int32),
            jax.ShapeDtypeStruct((1, E), jnp.int32),
        ],
        scratch_shapes=[
            pltpu.VMEM((1, E), jnp.float32),
            pltpu.VMEM((1, E), jnp.float32),
        ],
        compiler_params=pltpu.CompilerParams(
            dimension_semantics=("arbitrary", "arbitrary"),
        ),
    )(ep2d)
    dest = dest2d.reshape(npair)
    nblk = nblk2d.reshape(E)
    gsb = gsb2d.reshape(E)
    tokp = jnp.concatenate(
        [jnp.arange(T, dtype=jnp.int32), jnp.arange(T, dtype=jnp.int32)])

    mesh = plsc.VectorSubcoreMesh(core_axis_name="c", subcore_axis_name="s")
    dispatch = pl.kernel(
        _dispatch_body,
        mesh=mesh,
        out_type=jax.ShapeDtypeStruct((p_alloc, D), jnp.float32),
        scratch_types=[
            pltpu.VMEM((32,), jnp.int32),
            pltpu.VMEM((32,), jnp.int32),
            pltpu.VMEM((32, D), jnp.float32),
            pltpu.SemaphoreType.DMA,
            pltpu.SemaphoreType.DMA,
        ],
    )
    xs = dispatch(tokp, dest, x)

    bf = 2048
    nf = F // bf
    act = pl.pallas_call(
        _g1_body,
        grid_spec=pltpu.PrefetchScalarGridSpec(
            num_scalar_prefetch=2,
            grid=(E, nf, nbe),
            in_specs=[
                pl.BlockSpec(
                    (BT, D),
                    lambda e, f, j, nblk, gsb:
                    (gsb[e] + jnp.minimum(j, jnp.maximum(nblk[e] - 1, 0)),
                     0)),
                pl.BlockSpec((1, bf, D), lambda e, f, j, nblk, gsb:
                             (e, f, 0)),
                pl.BlockSpec((1, bf, D), lambda e, f, j, nblk, gsb:
                             (e, f, 0)),
            ],
            out_specs=pl.BlockSpec(
                (BT, bf),
                lambda e, f, j, nblk, gsb:
                (jnp.where(j < nblk[e], gsb[e] + j, trash), f)),
        ),
        out_shape=jax.ShapeDtypeStruct((p_alloc, F), jnp.float32),
        compiler_params=pltpu.CompilerParams(
            dimension_semantics=("arbitrary", "arbitrary", "arbitrary"),
            vmem_limit_bytes=63 * 1024 * 1024,
        ),
    )(nblk, gsb, xs, W1, W3)

    y = pl.pallas_call(
        _g2_body,
        grid_spec=pltpu.PrefetchScalarGridSpec(
            num_scalar_prefetch=2,
            grid=(E, nbe),
            in_specs=[
                pl.BlockSpec(
                    (BT, F),
                    lambda e, j, nblk, gsb:
                    (gsb[e] + jnp.minimum(j, jnp.maximum(nblk[e] - 1, 0)),
                     0)),
                pl.BlockSpec((1, D, F), lambda e, j, nblk, gsb: (e, 0, 0)),
            ],
            out_specs=pl.BlockSpec(
                (BT, D),
                lambda e, j, nblk, gsb:
                (jnp.where(j < nblk[e], gsb[e] + j, trash), 0)),
        ),
        out_shape=jax.ShapeDtypeStruct((p_alloc, D), jnp.float32),
        compiler_params=pltpu.CompilerParams(
            dimension_semantics=("arbitrary", "arbitrary"),
            vmem_limit_bytes=63 * 1024 * 1024,
        ),
    )(nblk, gsb, act, W2)

    combine = pl.kernel(
        _combine_body,
        mesh=mesh,
        out_type=jax.ShapeDtypeStruct((T, D), jnp.float32),
        scratch_types=[
            pltpu.VMEM((16,), jnp.int32),
            pltpu.VMEM((16,), jnp.int32),
            pltpu.VMEM((16, D), jnp.float32),
            pltpu.VMEM((16, D), jnp.float32),
            pltpu.VMEM((16, D), jnp.float32),
            pltpu.VMEM((16, D), jnp.float32),
            pltpu.VMEM((16, D), jnp.float32),
            pltpu.SemaphoreType.DMA,
            pltpu.SemaphoreType.DMA,
        ],
    )
    final = combine(y, dest[:T], dest[T:], wbe, wbo)
    return final.reshape(B, S, D), logits


# speedup vs baseline: 1.4168x; 1.0389x over previous
import functools

import jax
import jax.numpy as jnp
from jax import lax
from jax.experimental import pallas as pl
from jax.experimental.pallas import tpu as pltpu
from jax.experimental.pallas import tpu_sc as plsc

E = 8
TOPK = 2
BT = 256
NC = 2


def _router_body(x_ref, wg_ref, logits_ref, wbe_ref, wbo_ref, dest_ref,
                 nblk_ref, gsb_ref):
    logits = lax.dot_general(
        x_ref[...], wg_ref[...], (((1,), (1,)), ((), ())),
        preferred_element_type=jnp.float32)
    logits_ref[...] = logits
    m = jnp.max(logits, axis=-1, keepdims=True)
    ex = jnp.exp(logits - m)
    scores = ex / jnp.sum(ex, axis=-1, keepdims=True)
    cols = lax.broadcasted_iota(jnp.int32, scores.shape, 1)
    m1 = jnp.max(scores, axis=-1, keepdims=True)
    idx1 = jnp.min(jnp.where(scores == m1, cols, E), axis=-1, keepdims=True)
    masked = jnp.where(cols == idx1, -jnp.inf, scores)
    m2 = jnp.max(masked, axis=-1, keepdims=True)
    idx2 = jnp.min(jnp.where(masked == m2, cols, E), axis=-1, keepdims=True)
    denom = jnp.clip(m1 + m2, 1e-12, None)
    t, d = wbe_ref.shape
    wbe_ref[...] = lax.broadcast_in_dim(m1 / denom, (t, d), (0, 1))
    wbo_ref[...] = lax.broadcast_in_dim(m2 / denom, (t, d), (0, 1))

    oh = jnp.concatenate(
        [(cols == idx1).astype(jnp.float32),
         (cols == idx2).astype(jnp.float32)], axis=0)
    tot = jnp.sum(oh, axis=0, keepdims=True)
    padded = jnp.floor((tot + (BT - 1)) * (1.0 / BT)) * float(BT)
    ei = lax.broadcasted_iota(jnp.int32, (E, E), 0)
    ej = lax.broadcasted_iota(jnp.int32, (E, E), 1)
    excl = (ei < ej).astype(jnp.float32)
    gs = jnp.dot(padded, excl, preferred_element_type=jnp.float32)
    nblk_ref[...] = (padded * (1.0 / BT)).astype(jnp.int32)
    gsb_ref[...] = (gs * (1.0 / BT)).astype(jnp.int32)

    mbt = 1024
    npair = oh.shape[0]
    ri = lax.broadcasted_iota(jnp.int32, (mbt, mbt), 0)
    ci = lax.broadcasted_iota(jnp.int32, (mbt, mbt), 1)
    tri = (ri > ci).astype(jnp.float32)
    carry = gs
    for c in range(npair // mbt):
        ohc = oh[c * mbt:(c + 1) * mbt]
        rank = jnp.dot(tri, ohc, preferred_element_type=jnp.float32) + carry
        destc = jnp.sum(ohc * rank, axis=1, keepdims=True)
        dest_ref[c * mbt:(c + 1) * mbt, :] = destc.astype(jnp.int32)
        carry = carry + jnp.sum(ohc, axis=0, keepdims=True)


def _dispatch_body(tok_hbm, dest_hbm, x_hbm, xs_hbm, tok_v, dche_v, rows_v,
                   sem1, sem2):
    wid = lax.axis_index("s") * NC + lax.axis_index("c")
    pbase = wid * 128
    for k in range(4):
        pltpu.sync_copy(tok_hbm.at[pl.ds(pbase + k * 32, 32)], tok_v)
        pltpu.sync_copy(dest_hbm.at[pl.ds(pbase + k * 32, 32)], dche_v)
        pltpu.async_copy(x_hbm.at[tok_v], rows_v, sem1).wait()
        pltpu.async_copy(rows_v, xs_hbm.at[dche_v], sem2).wait()


def _g1_body(nblk_ref, gsb_ref, xs_ref, w1_ref, w3_ref, act_ref):
    e = pl.program_id(0)
    j = pl.program_id(2)

    @pl.when(j < nblk_ref[e])
    def _():
        xb = xs_ref[...]
        gate = lax.dot_general(
            xb, w1_ref[0], (((1,), (1,)), ((), ())),
            preferred_element_type=jnp.float32)
        up = lax.dot_general(
            xb, w3_ref[0], (((1,), (1,)), ((), ())),
            preferred_element_type=jnp.float32)
        act_ref[...] = gate * jax.nn.sigmoid(gate) * up


def _g2_body(nblk_ref, gsb_ref, act_ref, w2_ref, y_ref):
    e = pl.program_id(0)
    j = pl.program_id(1)

    @pl.when(j < nblk_ref[e])
    def _():
        y_ref[...] = lax.dot_general(
            act_ref[...], w2_ref[0], (((1,), (1,)), ((), ())),
            preferred_element_type=jnp.float32)


def _combine_body(y_hbm, de_hbm, do_hbm, wbe_hbm, wbo_hbm, out_hbm,
                  ie_v, io_v, r0_v, r1_v, w0_v, w1_v, acc_v, sem1, sem2):
    wid = lax.axis_index("s") * NC + lax.axis_index("c")
    t0base = wid * 64
    for g in range(4):
        t0 = t0base + g * 16
        pltpu.sync_copy(de_hbm.at[pl.ds(t0, 16)], ie_v)
        pltpu.sync_copy(do_hbm.at[pl.ds(t0, 16)], io_v)
        pltpu.async_copy(y_hbm.at[ie_v], r0_v, sem1).wait()
        pltpu.async_copy(y_hbm.at[io_v], r1_v, sem2).wait()
        pltpu.sync_copy(wbe_hbm.at[pl.ds(t0, 16)], w0_v)
        pltpu.sync_copy(wbo_hbm.at[pl.ds(t0, 16)], w1_v)
        for i in range(16):
            sw = pl.ds(0, 16)

            def vbody(vv, carry, i=i, sw=sw):
                s = pl.ds(vv * 16, 16)
                acc_v[i, s] = (w0_v[i, sw] * r0_v[i, s]
                               + w1_v[i, sw] * r1_v[i, s])
                return carry

            lax.fori_loop(0, 64, vbody, None)
        pltpu.sync_copy(acc_v, out_hbm.at[pl.ds(t0, 16)])


def kernel(hidden_states, W_gate, W1, W2, W3):
    B, S, D = hidden_states.shape
    T = B * S
    F = W1.shape[1]
    x = hidden_states.reshape(T, D)
    npair = T * TOPK
    nbe = T // BT
    nblk_max = npair // BT + E
    trash = nblk_max
    p_alloc = (nblk_max + 1) * BT

    logits, wbe, wbo, dest2d, nblk2d, gsb2d = pl.pallas_call(
        _router_body,
        out_shape=[
            jax.ShapeDtypeStruct((T, E), jnp.float32),
            jax.ShapeDtypeStruct((T, 128), jnp.float32),
            jax.ShapeDtypeStruct((T, 128), jnp.float32),
            jax.ShapeDtypeStruct((npair, 1), jnp.int32),
            jax.ShapeDtypeStruct((1, E), jnp.int32),
            jax.ShapeDtypeStruct((1, E), jnp.int32),
        ],
    )(x, W_gate)
    dest = dest2d.reshape(npair)
    nblk = nblk2d.reshape(E)
    gsb = gsb2d.reshape(E)
    tokp = jnp.concatenate(
        [jnp.arange(T, dtype=jnp.int32), jnp.arange(T, dtype=jnp.int32)])

    mesh = plsc.VectorSubcoreMesh(core_axis_name="c", subcore_axis_name="s")
    dispatch = pl.kernel(
        _dispatch_body,
        mesh=mesh,
        out_type=jax.ShapeDtypeStruct((p_alloc, D), jnp.float32),
        scratch_types=[
            pltpu.VMEM((32,), jnp.int32),
            pltpu.VMEM((32,), jnp.int32),
            pltpu.VMEM((32, D), jnp.float32),
            pltpu.SemaphoreType.DMA,
            pltpu.SemaphoreType.DMA,
        ],
    )
    xs = dispatch(tokp, dest, x)

    bf = 2048
    nf = F // bf
    act = pl.pallas_call(
        _g1_body,
        grid_spec=pltpu.PrefetchScalarGridSpec(
            num_scalar_prefetch=2,
            grid=(E, nf, nbe),
            in_specs=[
                pl.BlockSpec(
                    (BT, D),
                    lambda e, f, j, nblk, gsb:
                    (gsb[e] + jnp.minimum(j, jnp.maximum(nblk[e] - 1, 0)),
                     0)),
                pl.BlockSpec((1, bf, D), lambda e, f, j, nblk, gsb:
                             (e, f, 0)),
                pl.BlockSpec((1, bf, D), lambda e, f, j, nblk, gsb:
                             (e, f, 0)),
            ],
            out_specs=pl.BlockSpec(
                (BT, bf),
                lambda e, f, j, nblk, gsb:
                (jnp.where(j < nblk[e], gsb[e] + j, trash), f)),
        ),
        out_shape=jax.ShapeDtypeStruct((p_alloc, F), jnp.float32),
        compiler_params=pltpu.CompilerParams(
            dimension_semantics=("arbitrary", "arbitrary", "arbitrary"),
            vmem_limit_bytes=63 * 1024 * 1024,
        ),
    )(nblk, gsb, xs, W1, W3)

    y = pl.pallas_call(
        _g2_body,
        grid_spec=pltpu.PrefetchScalarGridSpec(
            num_scalar_prefetch=2,
            grid=(E, nbe),
            in_specs=[
                pl.BlockSpec(
                    (BT, F),
                    lambda e, j, nblk, gsb:
                    (gsb[e] + jnp.minimum(j, jnp.maximum(nblk[e] - 1, 0)),
                     0)),
                pl.BlockSpec((1, D, F), lambda e, j, nblk, gsb: (e, 0, 0)),
            ],
            out_specs=pl.BlockSpec(
                (BT, D),
                lambda e, j, nblk, gsb:
                (jnp.where(j < nblk[e], gsb[e] + j, trash), 0)),
        ),
        out_shape=jax.ShapeDtypeStruct((p_alloc, D), jnp.float32),
        compiler_params=pltpu.CompilerParams(
            dimension_semantics=("arbitrary", "arbitrary"),
            vmem_limit_bytes=63 * 1024 * 1024,
        ),
    )(nblk, gsb, act, W2)

    combine = pl.kernel(
        _combine_body,
        mesh=mesh,
        out_type=jax.ShapeDtypeStruct((T, D), jnp.float32),
        scratch_types=[
            pltpu.VMEM((16,), jnp.int32),
            pltpu.VMEM((16,), jnp.int32),
            pltpu.VMEM((16, D), jnp.float32),
            pltpu.VMEM((16, D), jnp.float32),
            pltpu.VMEM((16, 128), jnp.float32),
            pltpu.VMEM((16, 128), jnp.float32),
            pltpu.VMEM((16, D), jnp.float32),
            pltpu.SemaphoreType.DMA,
            pltpu.SemaphoreType.DMA,
        ],
    )
    final = combine(y, dest[:T], dest[T:], wbe, wbo)
    return final.reshape(B, S, D), logits


# speedup vs baseline: 1.4503x; 1.0236x over previous
import functools

import jax
import jax.numpy as jnp
from jax import lax
from jax.experimental import pallas as pl
from jax.experimental.pallas import tpu as pltpu
from jax.experimental.pallas import tpu_sc as plsc

E = 8
TOPK = 2
BT = 256
NC = 2


def _router_body(x_ref, wg_ref, logits_ref, wbe_ref, wbo_ref, dest_ref,
                 nblk_ref, gsb_ref):
    logits = lax.dot_general(
        x_ref[...], wg_ref[...], (((1,), (1,)), ((), ())),
        preferred_element_type=jnp.float32)
    logits_ref[...] = logits
    m = jnp.max(logits, axis=-1, keepdims=True)
    ex = jnp.exp(logits - m)
    scores = ex / jnp.sum(ex, axis=-1, keepdims=True)
    cols = lax.broadcasted_iota(jnp.int32, scores.shape, 1)
    m1 = jnp.max(scores, axis=-1, keepdims=True)
    idx1 = jnp.min(jnp.where(scores == m1, cols, E), axis=-1, keepdims=True)
    masked = jnp.where(cols == idx1, -jnp.inf, scores)
    m2 = jnp.max(masked, axis=-1, keepdims=True)
    idx2 = jnp.min(jnp.where(masked == m2, cols, E), axis=-1, keepdims=True)
    denom = jnp.clip(m1 + m2, 1e-12, None)
    t, d = wbe_ref.shape
    wbe_ref[...] = lax.broadcast_in_dim(m1 / denom, (t, d), (0, 1))
    wbo_ref[...] = lax.broadcast_in_dim(m2 / denom, (t, d), (0, 1))

    oh = jnp.concatenate(
        [(cols == idx1).astype(jnp.float32),
         (cols == idx2).astype(jnp.float32)], axis=0)
    tot = jnp.sum(oh, axis=0, keepdims=True)
    padded = jnp.floor((tot + (BT - 1)) * (1.0 / BT)) * float(BT)
    ei = lax.broadcasted_iota(jnp.int32, (E, E), 0)
    ej = lax.broadcasted_iota(jnp.int32, (E, E), 1)
    excl = (ei < ej).astype(jnp.float32)
    gs = jnp.dot(padded, excl, preferred_element_type=jnp.float32)
    nblk_ref[...] = (padded * (1.0 / BT)).astype(jnp.int32)
    gsb_ref[...] = (gs * (1.0 / BT)).astype(jnp.int32)

    mbt = 1024
    npair = oh.shape[0]
    ri = lax.broadcasted_iota(jnp.int32, (mbt, mbt), 0)
    ci = lax.broadcasted_iota(jnp.int32, (mbt, mbt), 1)
    tri = (ri > ci).astype(jnp.float32)
    carry = gs
    for c in range(npair // mbt):
        ohc = oh[c * mbt:(c + 1) * mbt]
        rank = jnp.dot(tri, ohc, preferred_element_type=jnp.float32) + carry
        destc = jnp.sum(ohc * rank, axis=1, keepdims=True)
        dest_ref[c * mbt:(c + 1) * mbt, :] = destc.astype(jnp.int32)
        carry = carry + jnp.sum(ohc, axis=0, keepdims=True)


def _dispatch_body(tok_hbm, dest_hbm, x_hbm, xs_hbm, tok_v, dche_v, rows_v,
                   sem1, sem2):
    wid = lax.axis_index("s") * NC + lax.axis_index("c")
    pbase = wid * 128
    for k in range(4):
        pltpu.sync_copy(tok_hbm.at[pl.ds(pbase + k * 32, 32)], tok_v)
        pltpu.sync_copy(dest_hbm.at[pl.ds(pbase + k * 32, 32)], dche_v)
        pltpu.async_copy(x_hbm.at[tok_v], rows_v, sem1).wait()
        pltpu.async_copy(rows_v, xs_hbm.at[dche_v], sem2).wait()


def _g1_body(nblk_ref, gsb_ref, xs_ref, w1_ref, w3_ref, act_ref):
    e = pl.program_id(0)
    j = pl.program_id(2)

    @pl.when(j < nblk_ref[e])
    def _():
        xb = xs_ref[...]
        gate = lax.dot_general(
            xb, w1_ref[0], (((1,), (1,)), ((), ())),
            preferred_element_type=jnp.float32)
        up = lax.dot_general(
            xb, w3_ref[0], (((1,), (1,)), ((), ())),
            preferred_element_type=jnp.float32)
        act_ref[...] = gate * jax.nn.sigmoid(gate) * up


def _g2_body(nblk_ref, gsb_ref, act_ref, w2_ref, y_ref):
    e = pl.program_id(0)
    j = pl.program_id(1)

    @pl.when(j < nblk_ref[e])
    def _():
        y_ref[...] = lax.dot_general(
            act_ref[...], w2_ref[0], (((1,), (1,)), ((), ())),
            preferred_element_type=jnp.float32)


def _combine_body(y_hbm, de_hbm, do_hbm, wbe_hbm, wbo_hbm, out_hbm,
                  ie_v, io_v, r0_v, r1_v, w0_v, w1_v, acc_v,
                  sa0, sa1, sb0, sb1):
    wid = lax.axis_index("s") * NC + lax.axis_index("c")
    t0base = wid * 64
    pend = {}
    for g in range(5):
        bi = g % 2
        if g < 4:
            t0 = t0base + g * 16
            pltpu.sync_copy(de_hbm.at[pl.ds(t0, 16)], ie_v.at[bi])
            pltpu.sync_copy(do_hbm.at[pl.ds(t0, 16)], io_v.at[bi])
            pltpu.sync_copy(wbe_hbm.at[pl.ds(t0, 16)], w0_v.at[bi])
            pltpu.sync_copy(wbo_hbm.at[pl.ds(t0, 16)], w1_v.at[bi])
            s0, s1 = ((sa0, sa1), (sb0, sb1))[bi]
            c0 = pltpu.async_copy(y_hbm.at[ie_v.at[bi]], r0_v.at[bi], s0)
            c1 = pltpu.async_copy(y_hbm.at[io_v.at[bi]], r1_v.at[bi], s1)
            pend[g] = (c0, c1)
        if g >= 1:
            gp = g - 1
            bp = gp % 2
            pend[gp][0].wait()
            pend[gp][1].wait()
            for i in range(16):
                sw = pl.ds(0, 16)

                def vbody(vv, carry, i=i, sw=sw, bp=bp):
                    s = pl.ds(vv * 16, 16)
                    acc_v[i, s] = (w0_v[bp, i, sw] * r0_v[bp, i, s]
                                   + w1_v[bp, i, sw] * r1_v[bp, i, s])
                    return carry

                lax.fori_loop(0, 64, vbody, None)
            pltpu.sync_copy(acc_v, out_hbm.at[pl.ds(t0base + gp * 16, 16)])


def kernel(hidden_states, W_gate, W1, W2, W3):
    B, S, D = hidden_states.shape
    T = B * S
    F = W1.shape[1]
    x = hidden_states.reshape(T, D)
    npair = T * TOPK
    nbe = T // BT
    nblk_max = npair // BT + E
    trash = nblk_max
    p_alloc = (nblk_max + 1) * BT

    logits, wbe, wbo, dest2d, nblk2d, gsb2d = pl.pallas_call(
        _router_body,
        out_shape=[
            jax.ShapeDtypeStruct((T, E), jnp.float32),
            jax.ShapeDtypeStruct((T, 128), jnp.float32),
            jax.ShapeDtypeStruct((T, 128), jnp.float32),
            jax.ShapeDtypeStruct((npair, 1), jnp.int32),
            jax.ShapeDtypeStruct((1, E), jnp.int32),
            jax.ShapeDtypeStruct((1, E), jnp.int32),
        ],
    )(x, W_gate)
    dest = dest2d.reshape(npair)
    nblk = nblk2d.reshape(E)
    gsb = gsb2d.reshape(E)
    tokp = jnp.concatenate(
        [jnp.arange(T, dtype=jnp.int32), jnp.arange(T, dtype=jnp.int32)])

    mesh = plsc.VectorSubcoreMesh(core_axis_name="c", subcore_axis_name="s")
    dispatch = pl.kernel(
        _dispatch_body,
        mesh=mesh,
        out_type=jax.ShapeDtypeStruct((p_alloc, D), jnp.float32),
        scratch_types=[
            pltpu.VMEM((32,), jnp.int32),
            pltpu.VMEM((32,), jnp.int32),
            pltpu.VMEM((32, D), jnp.float32),
            pltpu.SemaphoreType.DMA,
            pltpu.SemaphoreType.DMA,
        ],
    )
    xs = dispatch(tokp, dest, x)

    bf = 2048
    nf = F // bf
    act = pl.pallas_call(
        _g1_body,
        grid_spec=pltpu.PrefetchScalarGridSpec(
            num_scalar_prefetch=2,
            grid=(E, nf, nbe),
            in_specs=[
                pl.BlockSpec(
                    (BT, D),
                    lambda e, f, j, nblk, gsb:
                    (gsb[e] + jnp.minimum(j, jnp.maximum(nblk[e] - 1, 0)),
                     0)),
                pl.BlockSpec((1, bf, D), lambda e, f, j, nblk, gsb:
                             (e, f, 0)),
                pl.BlockSpec((1, bf, D), lambda e, f, j, nblk, gsb:
                             (e, f, 0)),
            ],
            out_specs=pl.BlockSpec(
                (BT, bf),
                lambda e, f, j, nblk, gsb:
                (jnp.where(j < nblk[e], gsb[e] + j, trash), f)),
        ),
        out_shape=jax.ShapeDtypeStruct((p_alloc, F), jnp.float32),
        compiler_params=pltpu.CompilerParams(
            dimension_semantics=("arbitrary", "arbitrary", "arbitrary"),
            vmem_limit_bytes=63 * 1024 * 1024,
        ),
    )(nblk, gsb, xs, W1, W3)

    y = pl.pallas_call(
        _g2_body,
        grid_spec=pltpu.PrefetchScalarGridSpec(
            num_scalar_prefetch=2,
            grid=(E, nbe),
            in_specs=[
                pl.BlockSpec(
                    (BT, F),
                    lambda e, j, nblk, gsb:
                    (gsb[e] + jnp.minimum(j, jnp.maximum(nblk[e] - 1, 0)),
                     0)),
                pl.BlockSpec((1, D, F), lambda e, j, nblk, gsb: (e, 0, 0)),
            ],
            out_specs=pl.BlockSpec(
                (BT, D),
                lambda e, j, nblk, gsb:
                (jnp.where(j < nblk[e], gsb[e] + j, trash), 0)),
        ),
        out_shape=jax.ShapeDtypeStruct((p_alloc, D), jnp.float32),
        compiler_params=pltpu.CompilerParams(
            dimension_semantics=("arbitrary", "arbitrary"),
            vmem_limit_bytes=63 * 1024 * 1024,
        ),
    )(nblk, gsb, act, W2)

    combine = pl.kernel(
        _combine_body,
        mesh=mesh,
        out_type=jax.ShapeDtypeStruct((T, D), jnp.float32),
        scratch_types=[
            pltpu.VMEM((2, 16), jnp.int32),
            pltpu.VMEM((2, 16), jnp.int32),
            pltpu.VMEM((2, 16, D), jnp.float32),
            pltpu.VMEM((2, 16, D), jnp.float32),
            pltpu.VMEM((2, 16, 128), jnp.float32),
            pltpu.VMEM((2, 16, 128), jnp.float32),
            pltpu.VMEM((16, D), jnp.float32),
            pltpu.SemaphoreType.DMA,
            pltpu.SemaphoreType.DMA,
            pltpu.SemaphoreType.DMA,
            pltpu.SemaphoreType.DMA,
        ],
    )
    final = combine(y, dest[:T], dest[T:], wbe, wbo)
    return final.reshape(B, S, D), logits


# speedup vs baseline: 1.4608x; 1.0072x over previous
import functools

import jax
import jax.numpy as jnp
from jax import lax
from jax.experimental import pallas as pl
from jax.experimental.pallas import tpu as pltpu
from jax.experimental.pallas import tpu_sc as plsc

E = 8
TOPK = 2
BT = 256
NC = 2


def _router_body(x_ref, wg_ref, logits_ref, wbe_ref, wbo_ref, dest_ref,
                 nblk_ref, gsb_ref):
    logits = lax.dot_general(
        x_ref[...], wg_ref[...], (((1,), (1,)), ((), ())),
        preferred_element_type=jnp.float32)
    logits_ref[...] = logits
    m = jnp.max(logits, axis=-1, keepdims=True)
    ex = jnp.exp(logits - m)
    scores = ex / jnp.sum(ex, axis=-1, keepdims=True)
    cols = lax.broadcasted_iota(jnp.int32, scores.shape, 1)
    m1 = jnp.max(scores, axis=-1, keepdims=True)
    idx1 = jnp.min(jnp.where(scores == m1, cols, E), axis=-1, keepdims=True)
    masked = jnp.where(cols == idx1, -jnp.inf, scores)
    m2 = jnp.max(masked, axis=-1, keepdims=True)
    idx2 = jnp.min(jnp.where(masked == m2, cols, E), axis=-1, keepdims=True)
    denom = jnp.clip(m1 + m2, 1e-12, None)
    t, d = wbe_ref.shape
    wbe_ref[...] = lax.broadcast_in_dim(m1 / denom, (t, d), (0, 1))
    wbo_ref[...] = lax.broadcast_in_dim(m2 / denom, (t, d), (0, 1))

    oh = jnp.concatenate(
        [(cols == idx1).astype(jnp.float32),
         (cols == idx2).astype(jnp.float32)], axis=0)
    tot = jnp.sum(oh, axis=0, keepdims=True)
    padded = jnp.floor((tot + (BT - 1)) * (1.0 / BT)) * float(BT)
    ei = lax.broadcasted_iota(jnp.int32, (E, E), 0)
    ej = lax.broadcasted_iota(jnp.int32, (E, E), 1)
    excl = (ei < ej).astype(jnp.float32)
    gs = jnp.dot(padded, excl, preferred_element_type=jnp.float32)
    nblk_ref[...] = (padded * (1.0 / BT)).astype(jnp.int32)
    gsb_ref[...] = (gs * (1.0 / BT)).astype(jnp.int32)

    mbt = 1024
    npair = oh.shape[0]
    ri = lax.broadcasted_iota(jnp.int32, (mbt, mbt), 0)
    ci = lax.broadcasted_iota(jnp.int32, (mbt, mbt), 1)
    tri = (ri > ci).astype(jnp.float32)
    carry = gs
    for c in range(npair // mbt):
        ohc = oh[c * mbt:(c + 1) * mbt]
        rank = jnp.dot(tri, ohc, preferred_element_type=jnp.float32) + carry
        destc = jnp.sum(ohc * rank, axis=1, keepdims=True)
        dest_ref[c * mbt:(c + 1) * mbt, :] = destc.astype(jnp.int32)
        carry = carry + jnp.sum(ohc, axis=0, keepdims=True)


def _dispatch_body(tok_hbm, dest_hbm, x_hbm, xs_hbm, tok_v, dche_v, rows_v,
                   sg0, sg1, ss0, ss1):
    wid = lax.axis_index("s") * NC + lax.axis_index("c")
    pbase = wid * 128
    pend_g = {}
    pend_s = {}
    for k in range(5):
        bi = k % 2
        if k < 4:
            if k >= 2:
                pend_s[k - 2].wait()
            pltpu.sync_copy(tok_hbm.at[pl.ds(pbase + k * 32, 32)],
                            tok_v.at[bi])
            pltpu.sync_copy(dest_hbm.at[pl.ds(pbase + k * 32, 32)],
                            dche_v.at[bi])
            pend_g[k] = pltpu.async_copy(x_hbm.at[tok_v.at[bi]],
                                         rows_v.at[bi], (sg0, sg1)[bi])
        if k >= 1:
            kp = k - 1
            bp = kp % 2
            pend_g[kp].wait()
            pend_s[kp] = pltpu.async_copy(rows_v.at[bp],
                                          xs_hbm.at[dche_v.at[bp]],
                                          (ss0, ss1)[bp])
    pend_s[2].wait()
    pend_s[3].wait()


def _g1_body(nblk_ref, gsb_ref, xs_ref, w1_ref, w3_ref, act_ref):
    e = pl.program_id(0)
    j = pl.program_id(2)

    @pl.when(j < nblk_ref[e])
    def _():
        xb = xs_ref[...]
        gate = lax.dot_general(
            xb, w1_ref[0], (((1,), (1,)), ((), ())),
            preferred_element_type=jnp.float32)
        up = lax.dot_general(
            xb, w3_ref[0], (((1,), (1,)), ((), ())),
            preferred_element_type=jnp.float32)
        act_ref[...] = gate * jax.nn.sigmoid(gate) * up


def _g2_body(nblk_ref, gsb_ref, act_ref, w2_ref, y_ref):
    e = pl.program_id(0)
    j = pl.program_id(1)

    @pl.when(j < nblk_ref[e])
    def _():
        y_ref[...] = lax.dot_general(
            act_ref[...], w2_ref[0], (((1,), (1,)), ((), ())),
            preferred_element_type=jnp.float32)


def _combine_body(y_hbm, de_hbm, do_hbm, wbe_hbm, wbo_hbm, out_hbm,
                  ie_v, io_v, r0_v, r1_v, w0_v, w1_v, acc_v,
                  sa0, sa1, sb0, sb1):
    wid = lax.axis_index("s") * NC + lax.axis_index("c")
    t0base = wid * 64
    pend = {}
    for g in range(5):
        bi = g % 2
        if g < 4:
            t0 = t0base + g * 16
            pltpu.sync_copy(de_hbm.at[pl.ds(t0, 16)], ie_v.at[bi])
            pltpu.sync_copy(do_hbm.at[pl.ds(t0, 16)], io_v.at[bi])
            pltpu.sync_copy(wbe_hbm.at[pl.ds(t0, 16)], w0_v.at[bi])
            pltpu.sync_copy(wbo_hbm.at[pl.ds(t0, 16)], w1_v.at[bi])
            s0, s1 = ((sa0, sa1), (sb0, sb1))[bi]
            c0 = pltpu.async_copy(y_hbm.at[ie_v.at[bi]], r0_v.at[bi], s0)
            c1 = pltpu.async_copy(y_hbm.at[io_v.at[bi]], r1_v.at[bi], s1)
            pend[g] = (c0, c1)
        if g >= 1:
            gp = g - 1
            bp = gp % 2
            pend[gp][0].wait()
            pend[gp][1].wait()
            for i in range(16):
                sw = pl.ds(0, 16)

                def vbody(vv, carry, i=i, sw=sw, bp=bp):
                    s = pl.ds(vv * 16, 16)
                    acc_v[i, s] = (w0_v[bp, i, sw] * r0_v[bp, i, s]
                                   + w1_v[bp, i, sw] * r1_v[bp, i, s])
                    return carry

                lax.fori_loop(0, 64, vbody, None)
            pltpu.sync_copy(acc_v, out_hbm.at[pl.ds(t0base + gp * 16, 16)])


def kernel(hidden_states, W_gate, W1, W2, W3):
    B, S, D = hidden_states.shape
    T = B * S
    F = W1.shape[1]
    x = hidden_states.reshape(T, D)
    npair = T * TOPK
    nbe = T // BT
    nblk_max = npair // BT + E
    trash = nblk_max
    p_alloc = (nblk_max + 1) * BT

    logits, wbe, wbo, dest2d, nblk2d, gsb2d = pl.pallas_call(
        _router_body,
        out_shape=[
            jax.ShapeDtypeStruct((T, E), jnp.float32),
            jax.ShapeDtypeStruct((T, 128), jnp.float32),
            jax.ShapeDtypeStruct((T, 128), jnp.float32),
            jax.ShapeDtypeStruct((npair, 1), jnp.int32),
            jax.ShapeDtypeStruct((1, E), jnp.int32),
            jax.ShapeDtypeStruct((1, E), jnp.int32),
        ],
    )(x, W_gate)
    dest = dest2d.reshape(npair)
    nblk = nblk2d.reshape(E)
    gsb = gsb2d.reshape(E)
    tokp = jnp.concatenate(
        [jnp.arange(T, dtype=jnp.int32), jnp.arange(T, dtype=jnp.int32)])

    mesh = plsc.VectorSubcoreMesh(core_axis_name="c", subcore_axis_name="s")
    dispatch = pl.kernel(
        _dispatch_body,
        mesh=mesh,
        out_type=jax.ShapeDtypeStruct((p_alloc, D), jnp.float32),
        scratch_types=[
            pltpu.VMEM((2, 32), jnp.int32),
            pltpu.VMEM((2, 32), jnp.int32),
            pltpu.VMEM((2, 32, D), jnp.float32),
            pltpu.SemaphoreType.DMA,
            pltpu.SemaphoreType.DMA,
            pltpu.SemaphoreType.DMA,
            pltpu.SemaphoreType.DMA,
        ],
    )
    xs = dispatch(tokp, dest, x)

    bf = 2048
    nf = F // bf
    act = pl.pallas_call(
        _g1_body,
        grid_spec=pltpu.PrefetchScalarGridSpec(
            num_scalar_prefetch=2,
            grid=(E, nf, nbe),
            in_specs=[
                pl.BlockSpec(
                    (BT, D),
                    lambda e, f, j, nblk, gsb:
                    (gsb[e] + jnp.minimum(j, jnp.maximum(nblk[e] - 1, 0)),
                     0)),
                pl.BlockSpec((1, bf, D), lambda e, f, j, nblk, gsb:
                             (e, f, 0)),
                pl.BlockSpec((1, bf, D), lambda e, f, j, nblk, gsb:
                             (e, f, 0)),
            ],
            out_specs=pl.BlockSpec(
                (BT, bf),
                lambda e, f, j, nblk, gsb:
                (jnp.where(j < nblk[e], gsb[e] + j, trash), f)),
        ),
        out_shape=jax.ShapeDtypeStruct((p_alloc, F), jnp.float32),
        compiler_params=pltpu.CompilerParams(
            dimension_semantics=("parallel", "parallel", "parallel"),
            vmem_limit_bytes=63 * 1024 * 1024,
        ),
    )(nblk, gsb, xs, W1, W3)

    y = pl.pallas_call(
        _g2_body,
        grid_spec=pltpu.PrefetchScalarGridSpec(
            num_scalar_prefetch=2,
            grid=(E, nbe),
            in_specs=[
                pl.BlockSpec(
                    (BT, F),
                    lambda e, j, nblk, gsb:
                    (gsb[e] + jnp.minimum(j, jnp.maximum(nblk[e] - 1, 0)),
                     0)),
                pl.BlockSpec((1, D, F), lambda e, j, nblk, gsb: (e, 0, 0)),
            ],
            out_specs=pl.BlockSpec(
                (BT, D),
                lambda e, j, nblk, gsb:
                (jnp.where(j < nblk[e], gsb[e] + j, trash), 0)),
        ),
        out_shape=jax.ShapeDtypeStruct((p_alloc, D), jnp.float32),
        compiler_params=pltpu.CompilerParams(
            dimension_semantics=("parallel", "parallel"),
            vmem_limit_bytes=63 * 1024 * 1024,
        ),
    )(nblk, gsb, act, W2)

    combine = pl.kernel(
        _combine_body,
        mesh=mesh,
        out_type=jax.ShapeDtypeStruct((T, D), jnp.float32),
        scratch_types=[
            pltpu.VMEM((2, 16), jnp.int32),
            pltpu.VMEM((2, 16), jnp.int32),
            pltpu.VMEM((2, 16, D), jnp.float32),
            pltpu.VMEM((2, 16, D), jnp.float32),
            pltpu.VMEM((2, 16, 128), jnp.float32),
            pltpu.VMEM((2, 16, 128), jnp.float32),
            pltpu.VMEM((16, D), jnp.float32),
            pltpu.SemaphoreType.DMA,
            pltpu.SemaphoreType.DMA,
            pltpu.SemaphoreType.DMA,
            pltpu.SemaphoreType.DMA,
        ],
    )
    final = combine(y, dest[:T], dest[T:], wbe, wbo)
    return final.reshape(B, S, D), logits
